# Initial kernel scaffold; baseline (speedup 1.0000x reference)
import jax
import jax.numpy as jnp
from jax.experimental import pallas as pl


def kernel(qk, bias):
    raise NotImplementedError("write your pallas kernel here")


# speedup vs baseline: 78.3183x; 78.3183x over previous
import jax
import jax.numpy as jnp
from jax.experimental import pallas as pl
from jax.experimental.pallas import tpu as pltpu

NQ = 2048
NK = 2048
NH = 16
BQ = 256
LW = NK + BQ + 128
RB_PAD = 4224


def _body(rb_ref, qk_ref, out_ref):
    qi = pl.program_id(1)
    base = (NQ - BQ) - qi * BQ
    w = rb_ref[0, 0, pl.ds(base, LW)]
    win = jnp.broadcast_to(w[None, :], (BQ, LW))
    rolled = pltpu.roll(win, LW - BQ, axis=1, stride=1, stride_axis=0)
    out_ref[0, 0] = qk_ref[0, 0] + rolled[:, :NK]


def kernel(qk, bias):
    rb = jnp.flip(bias, axis=0).T
    rb = jnp.pad(rb, ((0, 0), (0, RB_PAD - rb.shape[1])))
    rb = rb.reshape(NH, 1, RB_PAD)
    return pl.pallas_call(
        _body,
        grid=(NH, NQ // BQ),
        in_specs=[
            pl.BlockSpec((1, 1, RB_PAD), lambda h, qi: (h, 0, 0)),
            pl.BlockSpec((1, 1, BQ, NK), lambda h, qi: (0, h, qi, 0)),
        ],
        out_specs=pl.BlockSpec((1, 1, BQ, NK), lambda h, qi: (0, h, qi, 0)),
        out_shape=jax.ShapeDtypeStruct((1, NH, NQ, NK), jnp.float32),
    )(rb, qk)


# speedup vs baseline: 88.6596x; 1.1320x over previous
import jax
import jax.numpy as jnp
from jax.experimental import pallas as pl
from jax.experimental.pallas import tpu as pltpu

NQ = 2048
NK = 2048
NH = 16
BQ = 512
LW = NK + BQ
RB_PAD = 4224


def _body(rb_ref, qk_ref, out_ref):
    qi = pl.program_id(1)
    base = (NQ - BQ) - qi * BQ
    w = rb_ref[0, 0, pl.ds(base, LW)]
    win = jnp.broadcast_to(w[None, :], (BQ, LW))
    rolled = pltpu.roll(win, LW - BQ, axis=1, stride=1, stride_axis=0)
    out_ref[0, 0] = qk_ref[0, 0] + rolled[:, :NK]


def kernel(qk, bias):
    rb = jnp.flip(bias, axis=0).T
    rb = jnp.pad(rb, ((0, 0), (0, RB_PAD - rb.shape[1])))
    rb = rb.reshape(NH, 1, RB_PAD)
    return pl.pallas_call(
        _body,
        grid=(NH, NQ // BQ),
        in_specs=[
            pl.BlockSpec((1, 1, RB_PAD), lambda h, qi: (h, 0, 0)),
            pl.BlockSpec((1, 1, BQ, NK), lambda h, qi: (0, h, qi, 0)),
        ],
        out_specs=pl.BlockSpec((1, 1, BQ, NK), lambda h, qi: (0, h, qi, 0)),
        out_shape=jax.ShapeDtypeStruct((1, NH, NQ, NK), jnp.float32),
    )(rb, qk)


# speedup vs baseline: 89.6515x; 1.0112x over previous
import jax
import jax.numpy as jnp
from jax.experimental import pallas as pl
from jax.experimental.pallas import tpu as pltpu

NQ = 2048
NK = 2048
NH = 16
BQ = 1024
LW = NK + BQ
RB_PAD = 4224


def _body(rb_ref, qk_ref, out_ref):
    qi = pl.program_id(1)
    base = (NQ - BQ) - qi * BQ
    w = rb_ref[0, 0, pl.ds(base, LW)]
    win = jnp.broadcast_to(w[None, :], (BQ, LW))
    rolled = pltpu.roll(win, LW - BQ, axis=1, stride=1, stride_axis=0)
    out_ref[0, 0] = qk_ref[0, 0] + rolled[:, :NK]


def kernel(qk, bias):
    rb = jnp.flip(bias, axis=0).T
    rb = jnp.pad(rb, ((0, 0), (0, RB_PAD - rb.shape[1])))
    rb = rb.reshape(NH, 1, RB_PAD)
    return pl.pallas_call(
        _body,
        grid=(NH, NQ // BQ),
        in_specs=[
            pl.BlockSpec((1, 1, RB_PAD), lambda h, qi: (h, 0, 0)),
            pl.BlockSpec((1, 1, BQ, NK), lambda h, qi: (0, h, qi, 0)),
        ],
        out_specs=pl.BlockSpec((1, 1, BQ, NK), lambda h, qi: (0, h, qi, 0)),
        out_shape=jax.ShapeDtypeStruct((1, NH, NQ, NK), jnp.float32),
    )(rb, qk)
